# Initial kernel scaffold; baseline (speedup 1.0000x reference)
import jax
import jax.numpy as jnp
from jax.experimental import pallas as pl


def kernel(input, edge_index, A, B, gamma, beta):
    raise NotImplementedError("write your pallas kernel here")


# speedup vs baseline: 5.3152x; 5.3152x over previous
import functools

import jax
import jax.numpy as jnp
from jax import lax
from jax.experimental import pallas as pl
from jax.experimental.pallas import tpu as pltpu
from jax.experimental.pallas import tpu_sc as plsc

N_NODES = 10000
N_EDGES = 320000
F = 128
H = 64

NC = 2
NS = 16
L = 16
NW = NC * NS
EPW = N_EDGES // NW
CH = 80
NCHUNK = EPW // CH
RPS = 624
RCH = 208
TAIL = N_NODES - NS * RPS


def _quat_mat(w):
    r, i, j, k = jnp.split(w, 4, axis=1)
    r2 = jnp.concatenate([r, -i, -j, -k], axis=0)
    i2 = jnp.concatenate([i, r, -k, j], axis=0)
    j2 = jnp.concatenate([j, k, r, -i], axis=0)
    k2 = jnp.concatenate([k, -j, i, r], axis=0)
    return jnp.concatenate([r2, i2, j2, k2], axis=1)


def _support_body(x_ref, a_ref, b_ref, o_ref):
    a_h = _quat_mat(a_ref[...])
    b_h = _quat_mat(b_ref[...])
    zero = jnp.zeros((H, H), jnp.float32)
    w = jnp.concatenate(
        [jnp.concatenate([a_h, b_h], axis=1),
         jnp.concatenate([zero, a_h], axis=1)], axis=0)
    o_ref[...] = jnp.dot(x_ref[...], w, preferred_element_type=jnp.float32)


def _support(x, a, b):
    bn = 1000
    return pl.pallas_call(
        _support_body,
        grid=(N_NODES // bn,),
        in_specs=[
            pl.BlockSpec((bn, F), lambda i: (i, 0)),
            pl.BlockSpec((F // 8, H), lambda i: (0, 0)),
            pl.BlockSpec((F // 8, H), lambda i: (0, 0)),
        ],
        out_specs=pl.BlockSpec((bn, F), lambda i: (i, 0)),
        out_shape=jax.ShapeDtypeStruct((N_NODES, F), jnp.float32),
    )(x, a, b)


def _spmm_body(sup_hbm, src_hbm, dst_hbm, out_hbm,
               src_v, dst_v, rows_v, stage_v, acc_sh, sem):
    c = lax.axis_index("c")
    s = lax.axis_index("s")
    wid = c * NS + s

    def zrow(r, carry):
        for t in range(F // L):
            stage_v[r, pl.ds(t * L, L)] = jnp.zeros((L,), jnp.float32)
        return carry

    lax.fori_loop(0, RCH, zrow, 0)
    for j in range(RPS // RCH):
        pltpu.sync_copy(stage_v, acc_sh.at[pl.ds(s * RPS + j * RCH, RCH)])

    @pl.when(s == 0)
    def _():
        pltpu.sync_copy(stage_v.at[pl.ds(0, TAIL)],
                        acc_sh.at[pl.ds(NS * RPS, TAIL)])

    plsc.subcore_barrier()

    def body(i, carry):
        base = wid * EPW + i * CH
        pltpu.sync_copy(src_hbm.at[pl.ds(base, CH)], src_v)
        pltpu.sync_copy(dst_hbm.at[pl.ds(base, CH)], dst_v)
        pltpu.async_copy(sup_hbm.at[src_v], rows_v, sem).wait()
        pltpu.sync_copy(rows_v, acc_sh.at[dst_v], add=True)
        return carry

    lax.fori_loop(0, NCHUNK, body, 0)
    plsc.subcore_barrier()

    for j in range(RPS // RCH):
        r0 = s * RPS + j * RCH
        pltpu.sync_copy(acc_sh.at[pl.ds(r0, RCH)], stage_v)
        pltpu.sync_copy(stage_v, out_hbm.at[c, pl.ds(r0, RCH)])

    @pl.when(s == 0)
    def _():
        pltpu.sync_copy(acc_sh.at[pl.ds(NS * RPS, TAIL)],
                        stage_v.at[pl.ds(0, TAIL)])
        pltpu.sync_copy(stage_v.at[pl.ds(0, TAIL)],
                        out_hbm.at[c, pl.ds(NS * RPS, TAIL)])


def _spmm(support, src, dst):
    mesh = plsc.VectorSubcoreMesh(core_axis_name="c", subcore_axis_name="s")
    fn = functools.partial(
        pl.kernel,
        mesh=mesh,
        out_type=jax.ShapeDtypeStruct((NC, N_NODES, F), jnp.float32),
        scratch_types=[
            pltpu.VMEM((CH,), jnp.int32),
            pltpu.VMEM((CH,), jnp.int32),
            pltpu.VMEM((CH, F), jnp.float32),
            pltpu.VMEM((RCH, F), jnp.float32),
            pltpu.VMEM_SHARED((N_NODES, F), jnp.float32),
            pltpu.SemaphoreType.DMA,
        ],
    )(_spmm_body)
    return fn(support, src, dst)


def _stats_body(p_ref, y_ref, st_ref, acc_ref):
    i = pl.program_id(0)

    @pl.when(i == 0)
    def _():
        acc_ref[...] = jnp.zeros_like(acc_ref)

    y = p_ref[0] + p_ref[1]
    y_ref[...] = y
    acc_ref[0:1, :] += jnp.sum(y, axis=0, keepdims=True)
    acc_ref[1:2, :] += jnp.sum(y * y, axis=0, keepdims=True)

    @pl.when(i == pl.num_programs(0) - 1)
    def _():
        st_ref[...] = acc_ref[...]


def _stats(partials):
    bn = 1000
    return pl.pallas_call(
        _stats_body,
        grid=(N_NODES // bn,),
        in_specs=[pl.BlockSpec((NC, bn, F), lambda i: (0, i, 0))],
        out_specs=[
            pl.BlockSpec((bn, F), lambda i: (i, 0)),
            pl.BlockSpec((8, F), lambda i: (0, 0)),
        ],
        out_shape=[
            jax.ShapeDtypeStruct((N_NODES, F), jnp.float32),
            jax.ShapeDtypeStruct((8, F), jnp.float32),
        ],
        scratch_shapes=[pltpu.VMEM((8, F), jnp.float32)],
    )(partials)


def _norm_body(y_ref, st_ref, g_ref, b_ref, o_ref):
    n = jnp.float32(N_NODES)
    mean = st_ref[0:1, :] / n
    var = st_ref[1:2, :] / n - mean * mean
    rstd = lax.rsqrt(var + 1e-5)
    o_ref[...] = jnp.tanh((y_ref[...] - mean) * rstd * g_ref[...] + b_ref[...])


def _norm(y, st, gamma, beta):
    bn = 1000
    return pl.pallas_call(
        _norm_body,
        grid=(N_NODES // bn,),
        in_specs=[
            pl.BlockSpec((bn, F), lambda i: (i, 0)),
            pl.BlockSpec((8, F), lambda i: (0, 0)),
            pl.BlockSpec((1, F), lambda i: (0, 0)),
            pl.BlockSpec((1, F), lambda i: (0, 0)),
        ],
        out_specs=pl.BlockSpec((bn, F), lambda i: (i, 0)),
        out_shape=jax.ShapeDtypeStruct((N_NODES, F), jnp.float32),
    )(y, st, gamma.reshape(1, F), beta.reshape(1, F))


def kernel(input, edge_index, A, B, gamma, beta):
    x = input.astype(jnp.float32)
    ei = edge_index.astype(jnp.int32)
    dst = ei[0]
    src = ei[1]
    support = _support(x, A, B)
    partials = _spmm(support, src, dst)
    y, st = _stats(partials)
    return _norm(y, st, gamma, beta)


# speedup vs baseline: 9.5525x; 1.7972x over previous
import functools

import jax
import jax.numpy as jnp
from jax import lax
from jax.experimental import pallas as pl
from jax.experimental.pallas import tpu as pltpu
from jax.experimental.pallas import tpu_sc as plsc

N_NODES = 10000
N_EDGES = 320000
F = 128
H = 64

NC = 2
NS = 16
L = 16
NW = NC * NS
EPW = N_EDGES // NW
CH = 100
NCHUNK = EPW // CH
PASSES = 2
PCH = NCHUNK // PASSES
OUTER = PCH // 2
RPS = 624
TAIL = N_NODES - NS * RPS
WCHUNKS = tuple((j * 80, 80) for j in range(7)) + ((560, 64),)


def _quat_mat(w):
    r, i, j, k = jnp.split(w, 4, axis=1)
    r2 = jnp.concatenate([r, -i, -j, -k], axis=0)
    i2 = jnp.concatenate([i, r, -k, j], axis=0)
    j2 = jnp.concatenate([j, k, r, -i], axis=0)
    k2 = jnp.concatenate([k, -j, i, r], axis=0)
    return jnp.concatenate([r2, i2, j2, k2], axis=1)


def _support_body(x_ref, a_ref, b_ref, o_ref):
    a_h = _quat_mat(a_ref[...])
    b_h = _quat_mat(b_ref[...])
    zero = jnp.zeros((H, H), jnp.float32)
    w = jnp.concatenate(
        [jnp.concatenate([a_h, b_h], axis=1),
         jnp.concatenate([zero, a_h], axis=1)], axis=0)
    o_ref[...] = jnp.dot(x_ref[...], w, preferred_element_type=jnp.float32)


def _support(x, a, b):
    bn = 1000
    return pl.pallas_call(
        _support_body,
        grid=(N_NODES // bn,),
        in_specs=[
            pl.BlockSpec((bn, F), lambda i: (i, 0)),
            pl.BlockSpec((F // 8, H), lambda i: (0, 0)),
            pl.BlockSpec((F // 8, H), lambda i: (0, 0)),
        ],
        out_specs=pl.BlockSpec((bn, F), lambda i: (i, 0)),
        out_shape=jax.ShapeDtypeStruct((N_NODES, F), jnp.float32),
    )(x, a, b)


def _spmm_body(sup_hbm, src_hbm, dst_hbm, out_hbm,
               src_v, dst_v, rows0, rows1, acc_sh, g0, g1, s0, s1):
    c = lax.axis_index("c")
    s = lax.axis_index("s")
    wid = c * NS + s
    rows = (rows0, rows1)
    gsem = (g0, g1)
    ssem = (s0, s1)

    def _gather(i, b):
        return pltpu.async_copy(sup_hbm.at[src_v.at[i]], rows[b], gsem[b])

    def _scatter(i, b):
        return pltpu.async_copy(rows[b], acc_sh.at[dst_v.at[i]], ssem[b],
                                add=True)

    pltpu.sync_copy(src_hbm.at[wid, 0], src_v)
    pltpu.sync_copy(dst_hbm.at[wid, 0], dst_v)

    def zrow(r, carry):
        for t in range(F // L):
            rows0[r, pl.ds(t * L, L)] = jnp.zeros((L,), jnp.float32)
        return carry

    lax.fori_loop(0, CH, zrow, 0)
    for off, sz in WCHUNKS:
        pltpu.sync_copy(rows0.at[pl.ds(0, sz)],
                        acc_sh.at[pl.ds(s * RPS + off, sz)])

    @pl.when(s == 0)
    def _():
        pltpu.sync_copy(rows0.at[pl.ds(0, TAIL)],
                        acc_sh.at[pl.ds(NS * RPS, TAIL)])

    _gather(0, 0)
    plsc.subcore_barrier()

    def step(i, b, wait_guard, fill_guard):
        o = 1 - b
        pltpu.make_async_copy(sup_hbm.at[src_v.at[i]], rows[b],
                              gsem[b]).wait()
        _scatter(i, b)

        def _retire():
            pltpu.make_async_copy(rows[o], acc_sh.at[dst_v.at[i - 1]],
                                  ssem[o]).wait()

        if wait_guard is None:
            _retire()
        else:
            pl.when(wait_guard)(_retire)

        def _fill():
            _gather(i + 1, o)

        if fill_guard is None:
            _fill()
        else:
            pl.when(fill_guard)(_fill)

    def outer(k, carry):
        step(2 * k, 0, k > 0, None)
        step(2 * k + 1, 1, None, k < OUTER - 1)
        return carry

    for p in range(PASSES):
        if p > 0:
            pltpu.sync_copy(src_hbm.at[wid, p], src_v)
            pltpu.sync_copy(dst_hbm.at[wid, p], dst_v)
            _gather(0, 0)
        lax.fori_loop(0, OUTER, outer, 0)
        pltpu.make_async_copy(rows[1], acc_sh.at[dst_v.at[PCH - 1]],
                              ssem[1]).wait()

    plsc.subcore_barrier()

    for off, sz in WCHUNKS:
        r0 = s * RPS + off
        pltpu.sync_copy(acc_sh.at[pl.ds(r0, sz)], rows0.at[pl.ds(0, sz)])
        pltpu.sync_copy(rows0.at[pl.ds(0, sz)], out_hbm.at[c, pl.ds(r0, sz)])

    @pl.when(s == 0)
    def _():
        pltpu.sync_copy(acc_sh.at[pl.ds(NS * RPS, TAIL)],
                        rows1.at[pl.ds(0, TAIL)])
        pltpu.sync_copy(rows1.at[pl.ds(0, TAIL)],
                        out_hbm.at[c, pl.ds(NS * RPS, TAIL)])


def _spmm(support, src, dst):
    mesh = plsc.VectorSubcoreMesh(core_axis_name="c", subcore_axis_name="s")
    fn = functools.partial(
        pl.kernel,
        mesh=mesh,
        out_type=jax.ShapeDtypeStruct((NC, N_NODES, F), jnp.float32),
        scratch_types=[
            pltpu.VMEM((PCH, CH), jnp.int32),
            pltpu.VMEM((PCH, CH), jnp.int32),
            pltpu.VMEM((CH, F), jnp.float32),
            pltpu.VMEM((CH, F), jnp.float32),
            pltpu.VMEM_SHARED((N_NODES, F), jnp.float32),
            pltpu.SemaphoreType.DMA,
            pltpu.SemaphoreType.DMA,
            pltpu.SemaphoreType.DMA,
            pltpu.SemaphoreType.DMA,
        ],
    )(_spmm_body)
    return fn(support, src, dst)


def _stats_body(p_ref, y_ref, st_ref, acc_ref):
    i = pl.program_id(0)

    @pl.when(i == 0)
    def _():
        acc_ref[...] = jnp.zeros_like(acc_ref)

    y = p_ref[0] + p_ref[1]
    y_ref[...] = y
    acc_ref[0:1, :] += jnp.sum(y, axis=0, keepdims=True)
    acc_ref[1:2, :] += jnp.sum(y * y, axis=0, keepdims=True)

    @pl.when(i == pl.num_programs(0) - 1)
    def _():
        st_ref[...] = acc_ref[...]


def _stats(partials):
    bn = 1000
    return pl.pallas_call(
        _stats_body,
        grid=(N_NODES // bn,),
        in_specs=[pl.BlockSpec((NC, bn, F), lambda i: (0, i, 0))],
        out_specs=[
            pl.BlockSpec((bn, F), lambda i: (i, 0)),
            pl.BlockSpec((8, F), lambda i: (0, 0)),
        ],
        out_shape=[
            jax.ShapeDtypeStruct((N_NODES, F), jnp.float32),
            jax.ShapeDtypeStruct((8, F), jnp.float32),
        ],
        scratch_shapes=[pltpu.VMEM((8, F), jnp.float32)],
    )(partials)


def _norm_body(y_ref, st_ref, g_ref, b_ref, o_ref):
    n = jnp.float32(N_NODES)
    mean = st_ref[0:1, :] / n
    var = st_ref[1:2, :] / n - mean * mean
    rstd = lax.rsqrt(var + 1e-5)
    o_ref[...] = jnp.tanh((y_ref[...] - mean) * rstd * g_ref[...] + b_ref[...])


def _norm(y, st, gamma, beta):
    bn = 1000
    return pl.pallas_call(
        _norm_body,
        grid=(N_NODES // bn,),
        in_specs=[
            pl.BlockSpec((bn, F), lambda i: (i, 0)),
            pl.BlockSpec((8, F), lambda i: (0, 0)),
            pl.BlockSpec((1, F), lambda i: (0, 0)),
            pl.BlockSpec((1, F), lambda i: (0, 0)),
        ],
        out_specs=pl.BlockSpec((bn, F), lambda i: (i, 0)),
        out_shape=jax.ShapeDtypeStruct((N_NODES, F), jnp.float32),
    )(y, st, gamma.reshape(1, F), beta.reshape(1, F))


def kernel(input, edge_index, A, B, gamma, beta):
    x = input.astype(jnp.float32)
    ei = edge_index.astype(jnp.int32)
    dst = ei[0].reshape(NW, PASSES, PCH, CH)
    src = ei[1].reshape(NW, PASSES, PCH, CH)
    support = _support(x, A, B)
    partials = _spmm(support, src, dst)
    y, st = _stats(partials)
    return _norm(y, st, gamma, beta)


# speedup vs baseline: 9.7269x; 1.0182x over previous
import functools

import jax
import jax.numpy as jnp
from jax import lax
from jax.experimental import pallas as pl
from jax.experimental.pallas import tpu as pltpu
from jax.experimental.pallas import tpu_sc as plsc

N_NODES = 10000
N_EDGES = 320000
F = 128
H = 64

NC = 2
NS = 16
L = 16
NW = NC * NS
EPW = N_EDGES // NW
CH = 100
NCHUNK = EPW // CH
PASSES = 2
PCH = NCHUNK // PASSES
OUTER = PCH // 2
RPS = 624
TAIL = N_NODES - NS * RPS
WCHUNKS = tuple((j * 80, 80) for j in range(7)) + ((560, 64),)


def _quat_mat(w):
    r, i, j, k = jnp.split(w, 4, axis=1)
    r2 = jnp.concatenate([r, -i, -j, -k], axis=0)
    i2 = jnp.concatenate([i, r, -k, j], axis=0)
    j2 = jnp.concatenate([j, k, r, -i], axis=0)
    k2 = jnp.concatenate([k, -j, i, r], axis=0)
    return jnp.concatenate([r2, i2, j2, k2], axis=1)


def _spmm_body(sup_hbm, src_hbm, dst_hbm, out_hbm,
               src_v, dst_v, rows0, rows1, acc_sh, g0, g1, s0, s1):
    c = lax.axis_index("c")
    s = lax.axis_index("s")
    wid = c * NS + s
    rows = (rows0, rows1)
    gsem = (g0, g1)
    ssem = (s0, s1)

    def _gather(i, b):
        return pltpu.async_copy(sup_hbm.at[src_v.at[i]], rows[b], gsem[b])

    def _scatter(i, b):
        return pltpu.async_copy(rows[b], acc_sh.at[dst_v.at[i]], ssem[b],
                                add=True)

    pltpu.sync_copy(src_hbm.at[wid, 0], src_v)
    pltpu.sync_copy(dst_hbm.at[wid, 0], dst_v)

    def zrow(r, carry):
        for t in range(F // L):
            rows0[r, pl.ds(t * L, L)] = jnp.zeros((L,), jnp.float32)
        return carry

    lax.fori_loop(0, CH, zrow, 0)
    for off, sz in WCHUNKS:
        pltpu.sync_copy(rows0.at[pl.ds(0, sz)],
                        acc_sh.at[pl.ds(s * RPS + off, sz)])

    @pl.when(s == 0)
    def _():
        pltpu.sync_copy(rows0.at[pl.ds(0, TAIL)],
                        acc_sh.at[pl.ds(NS * RPS, TAIL)])

    _gather(0, 0)
    plsc.subcore_barrier()

    def step(i, b, wait_guard, fill_guard):
        o = 1 - b
        pltpu.make_async_copy(sup_hbm.at[src_v.at[i]], rows[b],
                              gsem[b]).wait()
        _scatter(i, b)

        def _retire():
            pltpu.make_async_copy(rows[o], acc_sh.at[dst_v.at[i - 1]],
                                  ssem[o]).wait()

        if wait_guard is None:
            _retire()
        else:
            pl.when(wait_guard)(_retire)

        def _fill():
            _gather(i + 1, o)

        if fill_guard is None:
            _fill()
        else:
            pl.when(fill_guard)(_fill)

    def outer(k, carry):
        step(2 * k, 0, k > 0, None)
        step(2 * k + 1, 1, None, k < OUTER - 1)
        return carry

    for p in range(PASSES):
        if p > 0:
            pltpu.sync_copy(src_hbm.at[wid, p], src_v)
            pltpu.sync_copy(dst_hbm.at[wid, p], dst_v)
            _gather(0, 0)
        lax.fori_loop(0, OUTER, outer, 0)
        pltpu.make_async_copy(rows[1], acc_sh.at[dst_v.at[PCH - 1]],
                              ssem[1]).wait()

    plsc.subcore_barrier()

    for off, sz in WCHUNKS:
        r0 = s * RPS + off
        pltpu.sync_copy(acc_sh.at[pl.ds(r0, sz)], rows0.at[pl.ds(0, sz)])
        pltpu.sync_copy(rows0.at[pl.ds(0, sz)], out_hbm.at[c, pl.ds(r0, sz)])

    @pl.when(s == 0)
    def _():
        pltpu.sync_copy(acc_sh.at[pl.ds(NS * RPS, TAIL)],
                        rows1.at[pl.ds(0, TAIL)])
        pltpu.sync_copy(rows1.at[pl.ds(0, TAIL)],
                        out_hbm.at[c, pl.ds(NS * RPS, TAIL)])


def _spmm(support, src, dst):
    mesh = plsc.VectorSubcoreMesh(core_axis_name="c", subcore_axis_name="s")
    fn = functools.partial(
        pl.kernel,
        mesh=mesh,
        out_type=jax.ShapeDtypeStruct((NC, N_NODES, F), jnp.float32),
        scratch_types=[
            pltpu.VMEM((PCH, CH), jnp.int32),
            pltpu.VMEM((PCH, CH), jnp.int32),
            pltpu.VMEM((CH, F), jnp.float32),
            pltpu.VMEM((CH, F), jnp.float32),
            pltpu.VMEM_SHARED((N_NODES, F), jnp.float32),
            pltpu.SemaphoreType.DMA,
            pltpu.SemaphoreType.DMA,
            pltpu.SemaphoreType.DMA,
            pltpu.SemaphoreType.DMA,
        ],
    )(_spmm_body)
    return fn(support, src, dst)


_BN = 1000
_G = N_NODES // _BN


def _post_body(p_ref, a_ref, b_ref, g_ref, bb_ref, o_ref, y_ref, acc_ref):
    ph = pl.program_id(0)
    i = pl.program_id(1)

    @pl.when(jnp.logical_and(ph == 0, i == 0))
    def _():
        acc_ref[...] = jnp.zeros_like(acc_ref)

    @pl.when(ph == 0)
    def _():
        a_h = _quat_mat(a_ref[...])
        b_h = _quat_mat(b_ref[...])
        zero = jnp.zeros((H, H), jnp.float32)
        w = jnp.concatenate(
            [jnp.concatenate([a_h, b_h], axis=1),
             jnp.concatenate([zero, a_h], axis=1)], axis=0)
        y = jnp.dot(p_ref[0] + p_ref[1], w,
                    preferred_element_type=jnp.float32,
                    precision=lax.Precision.HIGHEST)
        y_ref[pl.ds(i * _BN, _BN), :] = y
        acc_ref[0:1, :] += jnp.sum(y, axis=0, keepdims=True)
        acc_ref[1:2, :] += jnp.sum(y * y, axis=0, keepdims=True)
        o_ref[...] = y

    @pl.when(ph == 1)
    def _():
        n = jnp.float32(N_NODES)
        mean = acc_ref[0:1, :] / n
        var = acc_ref[1:2, :] / n - mean * mean
        rstd = lax.rsqrt(var + 1e-5)
        y = y_ref[pl.ds(i * _BN, _BN), :]
        o_ref[...] = jnp.tanh((y - mean) * rstd * g_ref[...] + bb_ref[...])


def _post(partials, a, b, gamma, beta):
    return pl.pallas_call(
        _post_body,
        grid=(2, _G),
        in_specs=[
            pl.BlockSpec((NC, _BN, F), lambda p, i: (0, i, 0)),
            pl.BlockSpec((F // 8, H), lambda p, i: (0, 0)),
            pl.BlockSpec((F // 8, H), lambda p, i: (0, 0)),
            pl.BlockSpec((1, F), lambda p, i: (0, 0)),
            pl.BlockSpec((1, F), lambda p, i: (0, 0)),
        ],
        out_specs=pl.BlockSpec((_BN, F), lambda p, i: (i, 0)),
        out_shape=jax.ShapeDtypeStruct((N_NODES, F), jnp.float32),
        scratch_shapes=[
            pltpu.VMEM((N_NODES, F), jnp.float32),
            pltpu.VMEM((8, F), jnp.float32),
        ],
    )(partials, a, b, gamma.reshape(1, F), beta.reshape(1, F))


def kernel(input, edge_index, A, B, gamma, beta):
    x = input.astype(jnp.float32)
    ei = edge_index.astype(jnp.int32)
    dst = ei[0].reshape(NW, PASSES, PCH, CH)
    src = ei[1].reshape(NW, PASSES, PCH, CH)
    partials = _spmm(x, src, dst)
    return _post(partials, A, B, gamma, beta)


# speedup vs baseline: 9.7962x; 1.0071x over previous
import functools

import jax
import jax.numpy as jnp
from jax import lax
from jax.experimental import pallas as pl
from jax.experimental.pallas import tpu as pltpu
from jax.experimental.pallas import tpu_sc as plsc

N_NODES = 10000
N_EDGES = 320000
F = 128
H = 64

NC = 2
NS = 16
L = 16
NW = NC * NS
EPW = N_EDGES // NW
CH = 100
NCHUNK = EPW // CH
PASSES = 2
PCH = NCHUNK // PASSES
OUTER = PCH // 2
RPS = 624
TAIL = N_NODES - NS * RPS
WCHUNKS = tuple((j * 80, 80) for j in range(7)) + ((560, 64),)


def _quat_mat(w):
    r, i, j, k = jnp.split(w, 4, axis=1)
    r2 = jnp.concatenate([r, -i, -j, -k], axis=0)
    i2 = jnp.concatenate([i, r, -k, j], axis=0)
    j2 = jnp.concatenate([j, k, r, -i], axis=0)
    k2 = jnp.concatenate([k, -j, i, r], axis=0)
    return jnp.concatenate([r2, i2, j2, k2], axis=1)


def _spmm_body(sup_hbm, src_hbm, dst_hbm, out_hbm,
               src_v, dst_v, rows0, rows1, acc_sh, g0, g1, s0, s1):
    c = lax.axis_index("c")
    s = lax.axis_index("s")
    wid = c * NS + s
    rows = (rows0, rows1)
    gsem = (g0, g1)
    ssem = (s0, s1)

    def _gather(i, b):
        return pltpu.async_copy(sup_hbm.at[src_v.at[i]], rows[b], gsem[b])

    def _scatter(i, b):
        return pltpu.async_copy(rows[b], acc_sh.at[dst_v.at[i]], ssem[b],
                                add=True)

    pltpu.sync_copy(src_hbm.at[wid, 0], src_v)
    pltpu.sync_copy(dst_hbm.at[wid, 0], dst_v)

    def zrow(r, carry):
        for t in range(F // L):
            rows0[r, pl.ds(t * L, L)] = jnp.zeros((L,), jnp.float32)
        return carry

    lax.fori_loop(0, CH, zrow, 0)
    for off, sz in WCHUNKS:
        pltpu.sync_copy(rows0.at[pl.ds(0, sz)],
                        acc_sh.at[pl.ds(s * RPS + off, sz)])

    @pl.when(s == 0)
    def _():
        pltpu.sync_copy(rows0.at[pl.ds(0, TAIL)],
                        acc_sh.at[pl.ds(NS * RPS, TAIL)])

    _gather(0, 0)
    plsc.subcore_barrier()

    def step(i, b, wait_guard, fill_guard):
        o = 1 - b
        pltpu.make_async_copy(sup_hbm.at[src_v.at[i]], rows[b],
                              gsem[b]).wait()
        _scatter(i, b)

        def _retire():
            pltpu.make_async_copy(rows[o], acc_sh.at[dst_v.at[i - 1]],
                                  ssem[o]).wait()

        if wait_guard is None:
            _retire()
        else:
            pl.when(wait_guard)(_retire)

        def _fill():
            _gather(i + 1, o)

        if fill_guard is None:
            _fill()
        else:
            pl.when(fill_guard)(_fill)

    def outer(k, carry):
        step(2 * k, 0, k > 0, None)
        step(2 * k + 1, 1, None, k < OUTER - 1)
        return carry

    for p in range(PASSES):
        if p > 0:
            pltpu.sync_copy(src_hbm.at[wid, p], src_v)
            pltpu.sync_copy(dst_hbm.at[wid, p], dst_v)
            _gather(0, 0)
        lax.fori_loop(0, OUTER, outer, 0)
        pltpu.make_async_copy(rows[1], acc_sh.at[dst_v.at[PCH - 1]],
                              ssem[1]).wait()

    plsc.subcore_barrier()

    pltpu.sync_copy(acc_sh.at[pl.ds(s * RPS, RPS)],
                    out_hbm.at[c, pl.ds(s * RPS, RPS)])

    @pl.when(s == 0)
    def _():
        pltpu.sync_copy(acc_sh.at[pl.ds(NS * RPS, TAIL)],
                        out_hbm.at[c, pl.ds(NS * RPS, TAIL)])


def _spmm(support, src, dst):
    mesh = plsc.VectorSubcoreMesh(core_axis_name="c", subcore_axis_name="s")
    fn = functools.partial(
        pl.kernel,
        mesh=mesh,
        out_type=jax.ShapeDtypeStruct((NC, N_NODES, F), jnp.float32),
        scratch_types=[
            pltpu.VMEM((PCH, CH), jnp.int32),
            pltpu.VMEM((PCH, CH), jnp.int32),
            pltpu.VMEM((CH, F), jnp.float32),
            pltpu.VMEM((CH, F), jnp.float32),
            pltpu.VMEM_SHARED((N_NODES, F), jnp.float32),
            pltpu.SemaphoreType.DMA,
            pltpu.SemaphoreType.DMA,
            pltpu.SemaphoreType.DMA,
            pltpu.SemaphoreType.DMA,
        ],
    )(_spmm_body)
    return fn(support, src, dst)


_BN = 1000
_G = N_NODES // _BN


def _post_body(p_ref, a_ref, b_ref, g_ref, bb_ref, o_ref, y_ref, acc_ref):
    ph = pl.program_id(0)
    i = pl.program_id(1)

    @pl.when(jnp.logical_and(ph == 0, i == 0))
    def _():
        acc_ref[...] = jnp.zeros_like(acc_ref)

    @pl.when(ph == 0)
    def _():
        a_h = _quat_mat(a_ref[...])
        b_h = _quat_mat(b_ref[...])
        zero = jnp.zeros((H, H), jnp.float32)
        w = jnp.concatenate(
            [jnp.concatenate([a_h, b_h], axis=1),
             jnp.concatenate([zero, a_h], axis=1)], axis=0)
        y = jnp.dot(p_ref[0] + p_ref[1], w,
                    preferred_element_type=jnp.float32,
                    precision=lax.Precision.HIGHEST)
        y_ref[pl.ds(i * _BN, _BN), :] = y
        acc_ref[0:1, :] += jnp.sum(y, axis=0, keepdims=True)
        acc_ref[1:2, :] += jnp.sum(y * y, axis=0, keepdims=True)
        o_ref[...] = y

    @pl.when(ph == 1)
    def _():
        n = jnp.float32(N_NODES)
        mean = acc_ref[0:1, :] / n
        var = acc_ref[1:2, :] / n - mean * mean
        rstd = lax.rsqrt(var + 1e-5)
        y = y_ref[pl.ds(i * _BN, _BN), :]
        o_ref[...] = jnp.tanh((y - mean) * rstd * g_ref[...] + bb_ref[...])


def _post(partials, a, b, gamma, beta):
    return pl.pallas_call(
        _post_body,
        grid=(2, _G),
        in_specs=[
            pl.BlockSpec((NC, _BN, F), lambda p, i: (0, i, 0)),
            pl.BlockSpec((F // 8, H), lambda p, i: (0, 0)),
            pl.BlockSpec((F // 8, H), lambda p, i: (0, 0)),
            pl.BlockSpec((1, F), lambda p, i: (0, 0)),
            pl.BlockSpec((1, F), lambda p, i: (0, 0)),
        ],
        out_specs=pl.BlockSpec((_BN, F), lambda p, i: (i, 0)),
        out_shape=jax.ShapeDtypeStruct((N_NODES, F), jnp.float32),
        scratch_shapes=[
            pltpu.VMEM((N_NODES, F), jnp.float32),
            pltpu.VMEM((8, F), jnp.float32),
        ],
    )(partials, a, b, gamma.reshape(1, F), beta.reshape(1, F))


def kernel(input, edge_index, A, B, gamma, beta):
    x = input.astype(jnp.float32)
    ei = edge_index.astype(jnp.int32)
    dst = ei[0].reshape(NW, PASSES, PCH, CH)
    src = ei[1].reshape(NW, PASSES, PCH, CH)
    partials = _spmm(x, src, dst)
    return _post(partials, A, B, gamma, beta)


# speedup vs baseline: 10.0723x; 1.0282x over previous
import functools

import jax
import jax.numpy as jnp
from jax import lax
from jax.experimental import pallas as pl
from jax.experimental.pallas import tpu as pltpu
from jax.experimental.pallas import tpu_sc as plsc

N_NODES = 10000
N_EDGES = 320000
F = 128
H = 64

NC = 2
NS = 16
L = 16
NW = NC * NS
EPW = N_EDGES // NW
CH = 100
NCHUNK = EPW // CH
PASSES = 2
PCH = NCHUNK // PASSES
OUTER = PCH // 2
RPS = 624
TAIL = N_NODES - NS * RPS
WCHUNKS = tuple((j * 80, 80) for j in range(7)) + ((560, 64),)


def _quat_mat(w):
    r, i, j, k = jnp.split(w, 4, axis=1)
    r2 = jnp.concatenate([r, -i, -j, -k], axis=0)
    i2 = jnp.concatenate([i, r, -k, j], axis=0)
    j2 = jnp.concatenate([j, k, r, -i], axis=0)
    k2 = jnp.concatenate([k, -j, i, r], axis=0)
    return jnp.concatenate([r2, i2, j2, k2], axis=1)


def _spmm_body(sup_hbm, src_hbm, dst_hbm, out_hbm,
               src_v, dst_v, rows0, rows1, acc_sh, g0, g1, s0, s1):
    c = lax.axis_index("c")
    s = lax.axis_index("s")
    wid = c * NS + s
    rows = (rows0, rows1)
    gsem = (g0, g1)
    ssem = (s0, s1)

    def _gather(i, b):
        return pltpu.async_copy(sup_hbm.at[src_v.at[i]], rows[b], gsem[b])

    def _scatter(i, b):
        return pltpu.async_copy(rows[b], acc_sh.at[dst_v.at[i]], ssem[b],
                                add=True)

    pltpu.sync_copy(src_hbm.at[wid, 0], src_v)
    pltpu.sync_copy(dst_hbm.at[wid, 0], dst_v)

    def zrow(r, carry):
        for t in range(F // L):
            rows0[r, pl.ds(t * L, L)] = jnp.zeros((L,), jnp.float32)
        return carry

    lax.fori_loop(0, CH, zrow, 0)
    for off, sz in WCHUNKS:
        pltpu.sync_copy(rows0.at[pl.ds(0, sz)],
                        acc_sh.at[pl.ds(s * RPS + off, sz)])

    @pl.when(s == 0)
    def _():
        pltpu.sync_copy(rows0.at[pl.ds(0, TAIL)],
                        acc_sh.at[pl.ds(NS * RPS, TAIL)])

    _gather(0, 0)
    plsc.subcore_barrier()

    def step(i, b, wait_guard, fill_guard):
        o = 1 - b
        pltpu.make_async_copy(sup_hbm.at[src_v.at[i]], rows[b],
                              gsem[b]).wait()
        _scatter(i, b)

        def _retire():
            pltpu.make_async_copy(rows[o], acc_sh.at[dst_v.at[i - 1]],
                                  ssem[o]).wait()

        if wait_guard is None:
            _retire()
        else:
            pl.when(wait_guard)(_retire)

        def _fill():
            _gather(i + 1, o)

        if fill_guard is None:
            _fill()
        else:
            pl.when(fill_guard)(_fill)

    def outer(k, carry):
        step(2 * k, 0, k > 0, None)
        step(2 * k + 1, 1, None, k < OUTER - 1)
        return carry

    for p in range(PASSES):
        if p > 0:
            pltpu.sync_copy(src_hbm.at[wid, p], src_v)
            pltpu.sync_copy(dst_hbm.at[wid, p], dst_v)
            _gather(0, 0)
        lax.fori_loop(0, OUTER, outer, 0)
        pltpu.make_async_copy(rows[1], acc_sh.at[dst_v.at[PCH - 1]],
                              ssem[1]).wait()

    plsc.subcore_barrier()

    pltpu.sync_copy(acc_sh.at[pl.ds(s * RPS, RPS)],
                    out_hbm.at[c, pl.ds(s * RPS, RPS)])

    @pl.when(s == 0)
    def _():
        pltpu.sync_copy(acc_sh.at[pl.ds(NS * RPS, TAIL)],
                        out_hbm.at[c, pl.ds(NS * RPS, TAIL)])


def _spmm(support, src, dst):
    mesh = plsc.VectorSubcoreMesh(core_axis_name="c", subcore_axis_name="s")
    fn = functools.partial(
        pl.kernel,
        mesh=mesh,
        out_type=jax.ShapeDtypeStruct((NC, N_NODES, F), jnp.float32),
        scratch_types=[
            pltpu.VMEM((PCH, CH), jnp.int32),
            pltpu.VMEM((PCH, CH), jnp.int32),
            pltpu.VMEM((CH, F), jnp.float32),
            pltpu.VMEM((CH, F), jnp.float32),
            pltpu.VMEM_SHARED((N_NODES, F), jnp.float32),
            pltpu.SemaphoreType.DMA,
            pltpu.SemaphoreType.DMA,
            pltpu.SemaphoreType.DMA,
            pltpu.SemaphoreType.DMA,
        ],
    )(_spmm_body)
    return fn(support, src, dst)


_BN = 1000
_G = N_NODES // _BN


def _post_body(p_ref, a_ref, b_ref, g_ref, bb_ref, o_ref, y_ref, acc_ref):
    ph = pl.program_id(0)
    i = pl.program_id(1)

    @pl.when(jnp.logical_and(ph == 0, i == 0))
    def _():
        acc_ref[...] = jnp.zeros_like(acc_ref)

    @pl.when(ph == 0)
    def _():
        a_h = _quat_mat(a_ref[...])
        b_h = _quat_mat(b_ref[...])
        zero = jnp.zeros((H, H), jnp.float32)
        w = jnp.concatenate(
            [jnp.concatenate([a_h, b_h], axis=1),
             jnp.concatenate([zero, a_h], axis=1)], axis=0)
        y = jnp.dot(p_ref[0] + p_ref[1], w,
                    preferred_element_type=jnp.float32,
                    precision=lax.Precision.HIGHEST)
        y_ref[pl.ds(i * _BN, _BN), :] = y
        acc_ref[0:1, :] += jnp.sum(y, axis=0, keepdims=True)
        acc_ref[1:2, :] += jnp.sum(y * y, axis=0, keepdims=True)

    @pl.when(ph == 1)
    def _():
        n = jnp.float32(N_NODES)
        mean = acc_ref[0:1, :] / n
        var = acc_ref[1:2, :] / n - mean * mean
        rstd = lax.rsqrt(var + 1e-5)
        y = y_ref[pl.ds(i * _BN, _BN), :]
        o_ref[...] = jnp.tanh((y - mean) * rstd * g_ref[...] + bb_ref[...])


def _post(partials, a, b, gamma, beta):
    return pl.pallas_call(
        _post_body,
        grid=(2, _G),
        in_specs=[
            pl.BlockSpec((NC, _BN, F),
                         lambda p, i: (0, jnp.where(p == 0, i, 0), 0)),
            pl.BlockSpec((F // 8, H), lambda p, i: (0, 0)),
            pl.BlockSpec((F // 8, H), lambda p, i: (0, 0)),
            pl.BlockSpec((1, F), lambda p, i: (0, 0)),
            pl.BlockSpec((1, F), lambda p, i: (0, 0)),
        ],
        out_specs=pl.BlockSpec((_BN, F),
                               lambda p, i: (jnp.where(p == 0, 0, i), 0)),
        out_shape=jax.ShapeDtypeStruct((N_NODES, F), jnp.float32),
        scratch_shapes=[
            pltpu.VMEM((N_NODES, F), jnp.float32),
            pltpu.VMEM((8, F), jnp.float32),
        ],
    )(partials, a, b, gamma.reshape(1, F), beta.reshape(1, F))


def kernel(input, edge_index, A, B, gamma, beta):
    x = input.astype(jnp.float32)
    ei = edge_index.astype(jnp.int32)
    dst = ei[0].reshape(NW, PASSES, PCH, CH)
    src = ei[1].reshape(NW, PASSES, PCH, CH)
    partials = _spmm(x, src, dst)
    return _post(partials, A, B, gamma, beta)
